# baseline (speedup 1.0000x reference)
import functools

import jax
import jax.numpy as jnp
from jax import lax
from jax.experimental import pallas as pl
from jax.experimental.pallas import tpu as pltpu
from jax.experimental.pallas import tpu_sc as plsc

B, C, T = 64, 8, 4096
N = 65536
E = 1048576
HID = 64
Q = HID // 4
NEW_T = 1024
FEAT = 32
EROWS = E // 128
NPT = N // 16

_MESH = plsc.VectorSubcoreMesh(
    core_axis_name="c", subcore_axis_name="s", num_cores=2, num_subcores=16)
_F32 = jnp.float32
_SC_PARAMS = pltpu.CompilerParams(use_tc_tiling_on_sc=False)


def _conv_body(x_ref, w1_ref, b1_ref, w2_ref, b2_ref, degA_ref, degB_ref,
               wg_ref, o0_ref, o1_ref, o2_ref, o3_ref):
    x4 = x_ref[0]

    def c1(p):
        acc = b1_ref[...]
        for k in range(5):
            m = p + k - 2
            r, s = m % 4, m // 4
            xs = x4[:, r, 1 + s:1 + s + NEW_T]
            acc = acc + jnp.dot(w1_ref[k], xs, preferred_element_type=_F32)
        return acc

    pe = jnp.maximum(jnp.maximum(c1(0), c1(1)), 0.0)
    po = jnp.maximum(jnp.maximum(c1(2), c1(3)), 0.0)
    z = jnp.zeros((16, 1), dtype=_F32)
    pep = jnp.concatenate([z, pe, z], axis=1)
    pop = jnp.concatenate([z, po, z], axis=1)

    def c2(parity):
        acc = b2_ref[...]
        for k in range(5):
            m = parity + k - 2
            r, s = m % 2, m // 2
            ph = pep if r == 0 else pop
            xs = ph[:, 1 + s:1 + s + NEW_T]
            acc = acc + jnp.dot(w2_ref[k], xs, preferred_element_type=_F32)
        return acc

    out = jnp.maximum(jnp.maximum(c2(0), c2(1)), 0.0)
    xt = out.T
    deg = degA_ref[...][:, 0:1] + degB_ref[...][:, 0:1] + 1.0
    h = jnp.dot(xt, wg_ref[...], preferred_element_type=_F32)
    h = h * lax.rsqrt(deg)
    o0_ref[...] = h[:, 0 * Q:1 * Q]
    o1_ref[...] = h[:, 1 * Q:2 * Q]
    o2_ref[...] = h[:, 2 * Q:3 * Q]
    o3_ref[...] = h[:, 3 * Q:4 * Q]


def _conv_call(x4, w1s, b1, w2s, b2, degA, degB, wg):
    return pl.pallas_call(
        _conv_body,
        grid=(B,),
        in_specs=[
            pl.BlockSpec((1, C, 4, 1026), lambda b: (b, 0, 0, 0)),
            pl.BlockSpec((5, 16, C), lambda b: (0, 0, 0)),
            pl.BlockSpec((16, 1), lambda b: (0, 0)),
            pl.BlockSpec((5, FEAT, 16), lambda b: (0, 0, 0)),
            pl.BlockSpec((FEAT, 1), lambda b: (0, 0)),
            pl.BlockSpec((NEW_T, 16), lambda b: (b, 0)),
            pl.BlockSpec((NEW_T, 16), lambda b: (b, 0)),
            pl.BlockSpec((FEAT, HID), lambda b: (0, 0)),
        ],
        out_specs=[pl.BlockSpec((NEW_T, Q), lambda b: (b, 0))] * 4,
        out_shape=[jax.ShapeDtypeStruct((N, Q), _F32)] * 4,
    )(x4, w1s, b1, w2s, b2, degA, degB, wg)


def _deg_body(dst_hbm, degA_hbm, degB_hbm, dstv, ones_v, zbuf, acc, sem):
    del sem
    c = lax.axis_index("c")
    s = lax.axis_index("s")
    for r in range(128):
        ones_v[r, :] = jnp.ones((16,), _F32)
        zbuf[r, :] = jnp.zeros((16,), _F32)

    @pl.loop(0, NPT // 128)
    def _zero(i):
        pltpu.sync_copy(zbuf, acc.at[pl.ds(s * NPT + i * 128, 128)])

    plsc.subcore_barrier()

    @pl.loop(0, 32)
    def _edges(it):
        row0 = c * (EROWS // 2) + s * 256 + it * 8
        pltpu.sync_copy(dst_hbm.at[pl.ds(row0, 8)], dstv)
        for j in range(8):
            pltpu.sync_copy(ones_v, acc.at[dstv.at[j]], add=True)

    plsc.subcore_barrier()

    @pl.when(c == 0)
    def _():
        pltpu.sync_copy(acc.at[pl.ds(s * NPT, NPT)],
                        degA_hbm.at[pl.ds(s * NPT, NPT)])

    @pl.when(c == 1)
    def _():
        pltpu.sync_copy(acc.at[pl.ds(s * NPT, NPT)],
                        degB_hbm.at[pl.ds(s * NPT, NPT)])


@functools.partial(
    pl.kernel,
    out_type=(jax.ShapeDtypeStruct((N, 16), _F32),
              jax.ShapeDtypeStruct((N, 16), _F32)),
    mesh=_MESH,
    scratch_types=[
        pltpu.VMEM((8, 128), jnp.int32),
        pltpu.VMEM((128, 16), _F32),
        pltpu.VMEM((128, 16), _F32),
        pltpu.VMEM_SHARED((N, 16), _F32),
        pltpu.SemaphoreType.DMA,
    ],
    compiler_params=_SC_PARAMS,
)
def _deg_call(dst_hbm, degA_hbm, degB_hbm, dstv, ones_v, zbuf, acc, sem):
    _deg_body(dst_hbm, degA_hbm, degB_hbm, dstv, ones_v, zbuf, acc, sem)


_SUP = 16


def _quarter_pass(h_hbm, s_hbm, ei_hbm, bufs, zbuf, acc, s):
    (ib0, isem0), (ib1, isem1), (msg0, sem0), (msg1, sem1) = bufs
    base = s * (EROWS // 16)

    def load_idx(sup, ib, isem):
        pltpu.async_copy(ei_hbm.at[pl.ds(base + sup * 32, 32)], ib, isem)

    def wait_idx(ib, isem):
        pltpu.make_async_copy(ei_hbm.at[pl.ds(base, 32)], ib, isem).wait()

    def issue_g(ib, ci, msg, sem):
        for j in range(8):
            pltpu.async_copy(h_hbm.at[ib.at[ci * 8 + j, 0]],
                             msg.at[pl.ds(j * 128, 128)], sem)

    def wait_g(ib, ci, msg, sem):
        for j in range(8):
            pltpu.make_async_copy(h_hbm.at[ib.at[ci * 8 + j, 0]],
                                  msg.at[pl.ds(j * 128, 128)], sem).wait()

    def scatter(ib, ci, msg, sem):
        copies = [pltpu.async_copy(msg.at[pl.ds(j * 128, 128)],
                                   acc.at[ib.at[ci * 8 + j, 1]], sem, add=True)
                  for j in range(8)]
        for cp in copies:
            cp.wait()

    @pl.loop(0, NPT // 128)
    def _zero(i):
        pltpu.sync_copy(zbuf, acc.at[pl.ds(s * NPT + i * 128, 128)])

    load_idx(0, ib0, isem0)
    wait_idx(ib0, isem0)
    load_idx(1, ib1, isem1)
    issue_g(ib0, 0, msg0, sem0)
    plsc.subcore_barrier()

    @pl.loop(0, _SUP // 2)
    def _edges(p):
        issue_g(ib0, 1, msg1, sem1)
        wait_g(ib0, 0, msg0, sem0)
        scatter(ib0, 0, msg0, sem0)
        issue_g(ib0, 2, msg0, sem0)
        wait_g(ib0, 1, msg1, sem1)
        scatter(ib0, 1, msg1, sem1)
        issue_g(ib0, 3, msg1, sem1)
        wait_g(ib0, 2, msg0, sem0)
        scatter(ib0, 2, msg0, sem0)
        wait_idx(ib1, isem1)
        issue_g(ib1, 0, msg0, sem0)
        wait_g(ib0, 3, msg1, sem1)
        scatter(ib0, 3, msg1, sem1)
        load_idx(lax.rem(2 * p + 2, _SUP), ib0, isem0)
        issue_g(ib1, 1, msg1, sem1)
        wait_g(ib1, 0, msg0, sem0)
        scatter(ib1, 0, msg0, sem0)
        issue_g(ib1, 2, msg0, sem0)
        wait_g(ib1, 1, msg1, sem1)
        scatter(ib1, 1, msg1, sem1)
        issue_g(ib1, 3, msg1, sem1)
        wait_g(ib1, 2, msg0, sem0)
        scatter(ib1, 2, msg0, sem0)
        wait_idx(ib0, isem0)
        issue_g(ib0, 0, msg0, sem0)
        wait_g(ib1, 3, msg1, sem1)
        scatter(ib1, 3, msg1, sem1)
        load_idx(lax.rem(2 * p + 3, _SUP), ib1, isem1)

    wait_g(ib0, 0, msg0, sem0)
    wait_idx(ib1, isem1)
    plsc.subcore_barrier()
    pltpu.sync_copy(acc.at[pl.ds(s * NPT, NPT)],
                    s_hbm.at[pl.ds(s * NPT, NPT)])


def _prop_body(h0_hbm, h1_hbm, h2_hbm, h3_hbm, ei_hbm,
               s0_hbm, s1_hbm, s2_hbm, s3_hbm,
               ib0, isem0, ib1, isem1, msg0, sem0, msg1, sem1,
               zbuf, acc):
    c = lax.axis_index("c")
    s = lax.axis_index("s")
    for r in range(128):
        zbuf[r, :] = jnp.zeros((16,), _F32)
    bufs = ((ib0, isem0), (ib1, isem1), (msg0, sem0), (msg1, sem1))

    @pl.when(c == 0)
    def _():
        _quarter_pass(h0_hbm, s0_hbm, ei_hbm, bufs, zbuf, acc, s)
        plsc.subcore_barrier()
        _quarter_pass(h1_hbm, s1_hbm, ei_hbm, bufs, zbuf, acc, s)

    @pl.when(c == 1)
    def _():
        _quarter_pass(h2_hbm, s2_hbm, ei_hbm, bufs, zbuf, acc, s)
        plsc.subcore_barrier()
        _quarter_pass(h3_hbm, s3_hbm, ei_hbm, bufs, zbuf, acc, s)


@functools.partial(
    pl.kernel,
    out_type=tuple(jax.ShapeDtypeStruct((N, Q), _F32) for _ in range(4)),
    mesh=_MESH,
    scratch_types=[
        pltpu.VMEM((32, 2, 128), jnp.int32),
        pltpu.SemaphoreType.DMA,
        pltpu.VMEM((32, 2, 128), jnp.int32),
        pltpu.SemaphoreType.DMA,
        pltpu.VMEM((1024, Q), _F32),
        pltpu.SemaphoreType.DMA,
        pltpu.VMEM((1024, Q), _F32),
        pltpu.SemaphoreType.DMA,
        pltpu.VMEM((128, Q), _F32),
        pltpu.VMEM_SHARED((N, Q), _F32),
    ],
    compiler_params=_SC_PARAMS,
)
def _prop_call(h0, h1, h2, h3, ei,
               s0, s1, s2, s3,
               ib0, isem0, ib1, isem1, msg0, sem0, msg1, sem1,
               zbuf, acc):
    _prop_body(h0, h1, h2, h3, ei, s0, s1, s2, s3,
               ib0, isem0, ib1, isem1, msg0, sem0, msg1, sem1,
               zbuf, acc)


def _dinv(degA_ref, degB_ref):
    deg = degA_ref[...][:, 0:1] + degB_ref[...][:, 0:1] + 1.0
    return lax.rsqrt(deg)


def _h2_body(s0_ref, s1_ref, s2_ref, s3_ref, h0_ref, h1_ref, h2_ref, h3_ref,
             degA_ref, degB_ref, b1_ref, w2_ref, *o_refs):
    dinv = _dinv(degA_ref, degB_ref)
    s1 = jnp.concatenate([s0_ref[...], s1_ref[...], s2_ref[...], s3_ref[...]],
                         axis=1)
    h1 = jnp.concatenate([h0_ref[...], h1_ref[...], h2_ref[...], h3_ref[...]],
                         axis=1)
    g1 = jnp.maximum(dinv * (s1 + h1) + b1_ref[...], 0.0)
    h2 = jnp.dot(g1, w2_ref[...], preferred_element_type=_F32) * dinv
    for q in range(4):
        o_refs[q][...] = h2[:, q * Q:(q + 1) * Q]


def _h2_call(sq, hq, degA, degB, b1, w2):
    blk = 1024
    return pl.pallas_call(
        _h2_body,
        grid=(N // blk,),
        in_specs=(
            [pl.BlockSpec((blk, Q), lambda i: (i, 0))] * 8
            + [pl.BlockSpec((blk, 16), lambda i: (i, 0))] * 2
            + [pl.BlockSpec((1, HID), lambda i: (0, 0)),
               pl.BlockSpec((HID, HID), lambda i: (0, 0))]
        ),
        out_specs=[pl.BlockSpec((blk, Q), lambda i: (i, 0))] * 4,
        out_shape=[jax.ShapeDtypeStruct((N, Q), _F32)] * 4,
    )(*sq, *hq, degA, degB, b1, w2)


def _out_body(s0_ref, s1_ref, s2_ref, s3_ref, h0_ref, h1_ref, h2_ref, h3_ref,
              degA_ref, degB_ref, b2_ref, cw_ref, cb_ref, o_ref):
    dinv = _dinv(degA_ref, degB_ref)
    s2 = jnp.concatenate([s0_ref[...], s1_ref[...], s2_ref[...], s3_ref[...]],
                         axis=1)
    h2 = jnp.concatenate([h0_ref[...], h1_ref[...], h2_ref[...], h3_ref[...]],
                         axis=1)
    g2 = jnp.maximum(dinv * (s2 + h2) + b2_ref[...], 0.0)
    pooled = jnp.mean(g2, axis=0, keepdims=True)
    o_ref[0] = jnp.dot(pooled, cw_ref[...],
                       preferred_element_type=_F32) + cb_ref[...]


def _out_call(sq, hq, degA, degB, b2, cw, cb):
    return pl.pallas_call(
        _out_body,
        grid=(B,),
        in_specs=(
            [pl.BlockSpec((NEW_T, Q), lambda b: (b, 0))] * 8
            + [pl.BlockSpec((NEW_T, 16), lambda b: (b, 0))] * 2
            + [pl.BlockSpec((1, HID), lambda b: (0, 0)),
               pl.BlockSpec((HID, 10), lambda b: (0, 0)),
               pl.BlockSpec((1, 10), lambda b: (0, 0))]
        ),
        out_specs=pl.BlockSpec((1, 1, 10), lambda b: (b, 0, 0)),
        out_shape=jax.ShapeDtypeStruct((B, 1, 10), _F32),
    )(*sq, *hq, degA, degB, b2, cw, cb).reshape(B, 10)


def kernel(x, edge_index, conv1_w, conv1_b, conv2_w, conv2_b,
           gcn1_w, gcn1_b, gcn2_w, gcn2_b, cls_w, cls_b):
    x4 = jnp.pad(x.reshape(B, C, NEW_T, 4).transpose(0, 1, 3, 2),
                 ((0, 0), (0, 0), (0, 0), (1, 1)))
    dst = edge_index[1].reshape(EROWS, 128)
    ei = jnp.stack([edge_index[0].reshape(EROWS, 128), dst], axis=1)
    w1s = jnp.transpose(conv1_w, (2, 0, 1))
    w2s = jnp.transpose(conv2_w, (2, 0, 1))

    degA, degB = _deg_call(dst)
    hq = _conv_call(x4, w1s, conv1_b.reshape(16, 1), w2s,
                    conv2_b.reshape(FEAT, 1), degA, degB, gcn1_w)
    sq = _prop_call(*hq, ei)
    h2q = _h2_call(sq, hq, degA, degB, gcn1_b.reshape(1, HID), gcn2_w)
    s2q = _prop_call(*h2q, ei)
    return _out_call(s2q, h2q, degA, degB,
                     gcn2_b.reshape(1, HID), cls_w, cls_b.reshape(1, 10))


# speedup vs baseline: 1.0109x; 1.0109x over previous
import functools

import jax
import jax.numpy as jnp
from jax import lax
from jax.experimental import pallas as pl
from jax.experimental.pallas import tpu as pltpu
from jax.experimental.pallas import tpu_sc as plsc

B, C, T = 64, 8, 4096
N = 65536
E = 1048576
HID = 64
Q = HID // 4
NEW_T = 1024
FEAT = 32
EROWS = E // 128
NPT = N // 16

_MESH = plsc.VectorSubcoreMesh(
    core_axis_name="c", subcore_axis_name="s", num_cores=2, num_subcores=16)
_F32 = jnp.float32
_SC_PARAMS = pltpu.CompilerParams(use_tc_tiling_on_sc=False)


def _conv_body(x_ref, w1_ref, b1_ref, w2_ref, b2_ref, degA_ref, degB_ref,
               wg_ref, o0_ref, o1_ref, o2_ref, o3_ref):
    x4 = x_ref[0]

    def c1(p):
        acc = b1_ref[...]
        for k in range(5):
            m = p + k - 2
            r, s = m % 4, m // 4
            xs = x4[:, r, 1 + s:1 + s + NEW_T]
            acc = acc + jnp.dot(w1_ref[k], xs, preferred_element_type=_F32)
        return acc

    pe = jnp.maximum(jnp.maximum(c1(0), c1(1)), 0.0)
    po = jnp.maximum(jnp.maximum(c1(2), c1(3)), 0.0)
    z = jnp.zeros((16, 1), dtype=_F32)
    pep = jnp.concatenate([z, pe, z], axis=1)
    pop = jnp.concatenate([z, po, z], axis=1)

    def c2(parity):
        acc = b2_ref[...]
        for k in range(5):
            m = parity + k - 2
            r, s = m % 2, m // 2
            ph = pep if r == 0 else pop
            xs = ph[:, 1 + s:1 + s + NEW_T]
            acc = acc + jnp.dot(w2_ref[k], xs, preferred_element_type=_F32)
        return acc

    out = jnp.maximum(jnp.maximum(c2(0), c2(1)), 0.0)
    xt = out.T
    deg = degA_ref[...][:, 0:1] + degB_ref[...][:, 0:1] + 1.0
    h = jnp.dot(xt, wg_ref[...], preferred_element_type=_F32)
    h = h * lax.rsqrt(deg)
    o0_ref[...] = h[:, 0 * Q:1 * Q]
    o1_ref[...] = h[:, 1 * Q:2 * Q]
    o2_ref[...] = h[:, 2 * Q:3 * Q]
    o3_ref[...] = h[:, 3 * Q:4 * Q]


def _conv_call(x4, w1s, b1, w2s, b2, degA, degB, wg):
    return pl.pallas_call(
        _conv_body,
        grid=(B,),
        in_specs=[
            pl.BlockSpec((1, C, 4, 1026), lambda b: (b, 0, 0, 0)),
            pl.BlockSpec((5, 16, C), lambda b: (0, 0, 0)),
            pl.BlockSpec((16, 1), lambda b: (0, 0)),
            pl.BlockSpec((5, FEAT, 16), lambda b: (0, 0, 0)),
            pl.BlockSpec((FEAT, 1), lambda b: (0, 0)),
            pl.BlockSpec((NEW_T, 16), lambda b: (b, 0)),
            pl.BlockSpec((NEW_T, 16), lambda b: (b, 0)),
            pl.BlockSpec((FEAT, HID), lambda b: (0, 0)),
        ],
        out_specs=[pl.BlockSpec((NEW_T, Q), lambda b: (b, 0))] * 4,
        out_shape=[jax.ShapeDtypeStruct((N, Q), _F32)] * 4,
    )(x4, w1s, b1, w2s, b2, degA, degB, wg)


def _deg_body(dst_hbm, degA_hbm, degB_hbm, dstv, ones_v, zbuf, acc, sem):
    c = lax.axis_index("c")
    s = lax.axis_index("s")
    for r in range(128):
        ones_v[r, :] = jnp.ones((16,), _F32)
        zbuf[r, :] = jnp.zeros((16,), _F32)

    @pl.loop(0, NPT // 128)
    def _zero(i):
        pltpu.sync_copy(zbuf, acc.at[pl.ds(s * NPT + i * 128, 128)])

    plsc.subcore_barrier()

    @pl.loop(0, 32)
    def _edges(it):
        row0 = c * (EROWS // 2) + s * 256 + it * 8
        pltpu.sync_copy(dst_hbm.at[pl.ds(row0, 8)], dstv)
        copies = [pltpu.async_copy(ones_v, acc.at[dstv.at[j]], sem, add=True)
                  for j in range(8)]
        for cp in copies:
            cp.wait()

    plsc.subcore_barrier()

    @pl.when(c == 0)
    def _():
        pltpu.sync_copy(acc.at[pl.ds(s * NPT, NPT)],
                        degA_hbm.at[pl.ds(s * NPT, NPT)])

    @pl.when(c == 1)
    def _():
        pltpu.sync_copy(acc.at[pl.ds(s * NPT, NPT)],
                        degB_hbm.at[pl.ds(s * NPT, NPT)])


@functools.partial(
    pl.kernel,
    out_type=(jax.ShapeDtypeStruct((N, 16), _F32),
              jax.ShapeDtypeStruct((N, 16), _F32)),
    mesh=_MESH,
    scratch_types=[
        pltpu.VMEM((8, 128), jnp.int32),
        pltpu.VMEM((128, 16), _F32),
        pltpu.VMEM((128, 16), _F32),
        pltpu.VMEM_SHARED((N, 16), _F32),
        pltpu.SemaphoreType.DMA,
    ],
    compiler_params=_SC_PARAMS,
)
def _deg_call(dst_hbm, degA_hbm, degB_hbm, dstv, ones_v, zbuf, acc, sem):
    _deg_body(dst_hbm, degA_hbm, degB_hbm, dstv, ones_v, zbuf, acc, sem)


_SUP = 16


def _quarter_pass(h_hbm, s_hbm, ei_hbm, bufs, zbuf, acc, s):
    (ib0, isem0), (ib1, isem1), (msg0, sem0), (msg1, sem1) = bufs
    base = s * (EROWS // 16)

    def load_idx(sup, ib, isem):
        pltpu.async_copy(ei_hbm.at[pl.ds(base + sup * 32, 32)], ib, isem)

    def wait_idx(ib, isem):
        pltpu.make_async_copy(ei_hbm.at[pl.ds(base, 32)], ib, isem).wait()

    def issue_g(ib, ci, msg, sem):
        for j in range(8):
            pltpu.async_copy(h_hbm.at[ib.at[ci * 8 + j, 0]],
                             msg.at[pl.ds(j * 128, 128)], sem)

    def wait_g(ib, ci, msg, sem):
        for j in range(8):
            pltpu.make_async_copy(h_hbm.at[ib.at[ci * 8 + j, 0]],
                                  msg.at[pl.ds(j * 128, 128)], sem).wait()

    def scatter(ib, ci, msg, sem):
        copies = [pltpu.async_copy(msg.at[pl.ds(j * 128, 128)],
                                   acc.at[ib.at[ci * 8 + j, 1]], sem, add=True)
                  for j in range(8)]
        for cp in copies:
            cp.wait()

    @pl.loop(0, NPT // 128)
    def _zero(i):
        pltpu.sync_copy(zbuf, acc.at[pl.ds(s * NPT + i * 128, 128)])

    load_idx(0, ib0, isem0)
    wait_idx(ib0, isem0)
    load_idx(1, ib1, isem1)
    issue_g(ib0, 0, msg0, sem0)
    plsc.subcore_barrier()

    @pl.loop(0, _SUP // 2)
    def _edges(p):
        issue_g(ib0, 1, msg1, sem1)
        wait_g(ib0, 0, msg0, sem0)
        scatter(ib0, 0, msg0, sem0)
        issue_g(ib0, 2, msg0, sem0)
        wait_g(ib0, 1, msg1, sem1)
        scatter(ib0, 1, msg1, sem1)
        issue_g(ib0, 3, msg1, sem1)
        wait_g(ib0, 2, msg0, sem0)
        scatter(ib0, 2, msg0, sem0)
        wait_idx(ib1, isem1)
        issue_g(ib1, 0, msg0, sem0)
        wait_g(ib0, 3, msg1, sem1)
        scatter(ib0, 3, msg1, sem1)
        load_idx(lax.rem(2 * p + 2, _SUP), ib0, isem0)
        issue_g(ib1, 1, msg1, sem1)
        wait_g(ib1, 0, msg0, sem0)
        scatter(ib1, 0, msg0, sem0)
        issue_g(ib1, 2, msg0, sem0)
        wait_g(ib1, 1, msg1, sem1)
        scatter(ib1, 1, msg1, sem1)
        issue_g(ib1, 3, msg1, sem1)
        wait_g(ib1, 2, msg0, sem0)
        scatter(ib1, 2, msg0, sem0)
        wait_idx(ib0, isem0)
        issue_g(ib0, 0, msg0, sem0)
        wait_g(ib1, 3, msg1, sem1)
        scatter(ib1, 3, msg1, sem1)
        load_idx(lax.rem(2 * p + 3, _SUP), ib1, isem1)

    wait_g(ib0, 0, msg0, sem0)
    wait_idx(ib1, isem1)
    plsc.subcore_barrier()
    pltpu.sync_copy(acc.at[pl.ds(s * NPT, NPT)],
                    s_hbm.at[pl.ds(s * NPT, NPT)])


def _prop_body(h0_hbm, h1_hbm, h2_hbm, h3_hbm, ei_hbm,
               s0_hbm, s1_hbm, s2_hbm, s3_hbm,
               ib0, isem0, ib1, isem1, msg0, sem0, msg1, sem1,
               zbuf, acc):
    c = lax.axis_index("c")
    s = lax.axis_index("s")
    for r in range(128):
        zbuf[r, :] = jnp.zeros((16,), _F32)
    bufs = ((ib0, isem0), (ib1, isem1), (msg0, sem0), (msg1, sem1))

    @pl.when(c == 0)
    def _():
        _quarter_pass(h0_hbm, s0_hbm, ei_hbm, bufs, zbuf, acc, s)
        plsc.subcore_barrier()
        _quarter_pass(h1_hbm, s1_hbm, ei_hbm, bufs, zbuf, acc, s)

    @pl.when(c == 1)
    def _():
        _quarter_pass(h2_hbm, s2_hbm, ei_hbm, bufs, zbuf, acc, s)
        plsc.subcore_barrier()
        _quarter_pass(h3_hbm, s3_hbm, ei_hbm, bufs, zbuf, acc, s)


@functools.partial(
    pl.kernel,
    out_type=tuple(jax.ShapeDtypeStruct((N, Q), _F32) for _ in range(4)),
    mesh=_MESH,
    scratch_types=[
        pltpu.VMEM((32, 2, 128), jnp.int32),
        pltpu.SemaphoreType.DMA,
        pltpu.VMEM((32, 2, 128), jnp.int32),
        pltpu.SemaphoreType.DMA,
        pltpu.VMEM((1024, Q), _F32),
        pltpu.SemaphoreType.DMA,
        pltpu.VMEM((1024, Q), _F32),
        pltpu.SemaphoreType.DMA,
        pltpu.VMEM((128, Q), _F32),
        pltpu.VMEM_SHARED((N, Q), _F32),
    ],
    compiler_params=_SC_PARAMS,
)
def _prop_call(h0, h1, h2, h3, ei,
               s0, s1, s2, s3,
               ib0, isem0, ib1, isem1, msg0, sem0, msg1, sem1,
               zbuf, acc):
    _prop_body(h0, h1, h2, h3, ei, s0, s1, s2, s3,
               ib0, isem0, ib1, isem1, msg0, sem0, msg1, sem1,
               zbuf, acc)


def _dinv(degA_ref, degB_ref):
    deg = degA_ref[...][:, 0:1] + degB_ref[...][:, 0:1] + 1.0
    return lax.rsqrt(deg)


def _h2_body(s0_ref, s1_ref, s2_ref, s3_ref, h0_ref, h1_ref, h2_ref, h3_ref,
             degA_ref, degB_ref, b1_ref, w2_ref, *o_refs):
    dinv = _dinv(degA_ref, degB_ref)
    s1 = jnp.concatenate([s0_ref[...], s1_ref[...], s2_ref[...], s3_ref[...]],
                         axis=1)
    h1 = jnp.concatenate([h0_ref[...], h1_ref[...], h2_ref[...], h3_ref[...]],
                         axis=1)
    g1 = jnp.maximum(dinv * (s1 + h1) + b1_ref[...], 0.0)
    h2 = jnp.dot(g1, w2_ref[...], preferred_element_type=_F32) * dinv
    for q in range(4):
        o_refs[q][...] = h2[:, q * Q:(q + 1) * Q]


def _h2_call(sq, hq, degA, degB, b1, w2):
    blk = 1024
    return pl.pallas_call(
        _h2_body,
        grid=(N // blk,),
        in_specs=(
            [pl.BlockSpec((blk, Q), lambda i: (i, 0))] * 8
            + [pl.BlockSpec((blk, 16), lambda i: (i, 0))] * 2
            + [pl.BlockSpec((1, HID), lambda i: (0, 0)),
               pl.BlockSpec((HID, HID), lambda i: (0, 0))]
        ),
        out_specs=[pl.BlockSpec((blk, Q), lambda i: (i, 0))] * 4,
        out_shape=[jax.ShapeDtypeStruct((N, Q), _F32)] * 4,
    )(*sq, *hq, degA, degB, b1, w2)


def _out_body(s0_ref, s1_ref, s2_ref, s3_ref, h0_ref, h1_ref, h2_ref, h3_ref,
              degA_ref, degB_ref, b2_ref, cw_ref, cb_ref, o_ref):
    dinv = _dinv(degA_ref, degB_ref)
    s2 = jnp.concatenate([s0_ref[...], s1_ref[...], s2_ref[...], s3_ref[...]],
                         axis=1)
    h2 = jnp.concatenate([h0_ref[...], h1_ref[...], h2_ref[...], h3_ref[...]],
                         axis=1)
    g2 = jnp.maximum(dinv * (s2 + h2) + b2_ref[...], 0.0)
    pooled = jnp.mean(g2, axis=0, keepdims=True)
    o_ref[0] = jnp.dot(pooled, cw_ref[...],
                       preferred_element_type=_F32) + cb_ref[...]


def _out_call(sq, hq, degA, degB, b2, cw, cb):
    return pl.pallas_call(
        _out_body,
        grid=(B,),
        in_specs=(
            [pl.BlockSpec((NEW_T, Q), lambda b: (b, 0))] * 8
            + [pl.BlockSpec((NEW_T, 16), lambda b: (b, 0))] * 2
            + [pl.BlockSpec((1, HID), lambda b: (0, 0)),
               pl.BlockSpec((HID, 10), lambda b: (0, 0)),
               pl.BlockSpec((1, 10), lambda b: (0, 0))]
        ),
        out_specs=pl.BlockSpec((1, 1, 10), lambda b: (b, 0, 0)),
        out_shape=jax.ShapeDtypeStruct((B, 1, 10), _F32),
    )(*sq, *hq, degA, degB, b2, cw, cb).reshape(B, 10)


def kernel(x, edge_index, conv1_w, conv1_b, conv2_w, conv2_b,
           gcn1_w, gcn1_b, gcn2_w, gcn2_b, cls_w, cls_b):
    x4 = jnp.pad(x.reshape(B, C, NEW_T, 4).transpose(0, 1, 3, 2),
                 ((0, 0), (0, 0), (0, 0), (1, 1)))
    dst = edge_index[1].reshape(EROWS, 128)
    ei = jnp.stack([edge_index[0].reshape(EROWS, 128), dst], axis=1)
    w1s = jnp.transpose(conv1_w, (2, 0, 1))
    w2s = jnp.transpose(conv2_w, (2, 0, 1))

    degA, degB = _deg_call(dst)
    hq = _conv_call(x4, w1s, conv1_b.reshape(16, 1), w2s,
                    conv2_b.reshape(FEAT, 1), degA, degB, gcn1_w)
    sq = _prop_call(*hq, ei)
    h2q = _h2_call(sq, hq, degA, degB, gcn1_b.reshape(1, HID), gcn2_w)
    s2q = _prop_call(*h2q, ei)
    return _out_call(s2q, h2q, degA, degB,
                     gcn2_b.reshape(1, HID), cls_w, cls_b.reshape(1, 10))
